# baseline (speedup 1.0000x reference)
import dataclasses
import functools

import jax
import jax.numpy as jnp
import numpy as np
from jax.experimental import pallas as pl
from jax.experimental.pallas import tpu as pltpu
from jax.experimental.pallas import tpu_sc as plsc

_W = 1024
_L = 16


@functools.cache
def _make_lookup(B, V, D, dtype, scale):
    mesh = plsc.VectorSubcoreMesh(core_axis_name="core", subcore_axis_name="subcore")
    cp = pltpu.CompilerParams()
    if "needs_layout_passes" in pltpu.CompilerParams.__dataclass_fields__:
        cp = dataclasses.replace(cp, needs_layout_passes=False)

    @functools.partial(
        pl.kernel,
        out_type=jax.ShapeDtypeStruct((B * D,), dtype),
        mesh=mesh,
        scratch_types=[pltpu.VMEM((V * D,), dtype)],
        compiler_params=cp,
    )
    def lookup(table_hbm, idx_hbm, out_hbm, tab_vmem):
        pltpu.sync_copy(table_hbm, tab_vmem)

        @pl.loop(0, V * D // _L)
        def _(i):
            sl = pl.ds(i * _L, _L)
            tab_vmem[sl] = tab_vmem[sl] * scale

        lanes_d = jax.lax.iota(jnp.int32, _L) * D

        def body(i_vmem, o_vmem):
            @pl.loop(0, _W // _L)
            def _(g):
                idxv = i_vmem[0, pl.ds(g * _L, _L)]
                src = idxv * D
                dst = lanes_d + g * (_L * D)
                for j in range(D):
                    vals = plsc.load_gather(tab_vmem, [src + j])
                    plsc.store_scatter(o_vmem, [dst + j], vals)

        pltpu.emit_pipeline(
            body,
            grid=(B // _W,),
            in_specs=[pl.BlockSpec((1, _W), index_map=lambda i: (0, i))],
            out_specs=[pl.BlockSpec((_W * D,), index_map=lambda i: (i,))],
            core_axis_name=("core", "subcore"),
            dimension_semantics=(pltpu.PARALLEL,),
        )(idx_hbm, out_hbm)

    return lookup


def kernel(x, table):
    V, D = table.shape
    B = x.size
    scale = float(np.sqrt(D).astype(np.float32))
    idx = x.reshape(1, B)
    out = _make_lookup(B, V, D, table.dtype, scale)(table.reshape(V * D), idx)
    return out.reshape(*x.shape, D)


# speedup vs baseline: 1.2523x; 1.2523x over previous
import dataclasses
import functools

import jax
import jax.numpy as jnp
import numpy as np
from jax.experimental import pallas as pl
from jax.experimental.pallas import tpu as pltpu
from jax.experimental.pallas import tpu_sc as plsc

_W = 1024
_L = 16


@functools.cache
def _make_lookup(B, V, D, dtype, scale):
    mesh = plsc.VectorSubcoreMesh(core_axis_name="core", subcore_axis_name="subcore")
    cp = pltpu.CompilerParams()
    if "needs_layout_passes" in pltpu.CompilerParams.__dataclass_fields__:
        cp = dataclasses.replace(cp, needs_layout_passes=False)

    @functools.partial(
        pl.kernel,
        out_type=jax.ShapeDtypeStruct((B * D,), dtype),
        mesh=mesh,
        scratch_types=[pltpu.VMEM((V * D,), dtype)],
        compiler_params=cp,
    )
    def lookup(table_hbm, idx_hbm, out_hbm, tab_vmem):
        pltpu.sync_copy(table_hbm, tab_vmem)

        @pl.loop(0, V * D // _L)
        def _(i):
            sl = pl.ds(i * _L, _L)
            tab_vmem[sl] = tab_vmem[sl] * scale

        lanes_d = jax.lax.iota(jnp.int32, _L) * D

        def body(i_vmem, o_vmem):
            @plsc.parallel_loop(0, _W // _L, unroll=4)
            def _(g):
                idxv = i_vmem[0, pl.ds(g * _L, _L)]
                src = idxv * D
                dst = lanes_d + g * (_L * D)
                for j in range(D):
                    vals = plsc.load_gather(tab_vmem, [src + j])
                    plsc.store_scatter(o_vmem, [dst + j], vals)

        pltpu.emit_pipeline(
            body,
            grid=(B // _W,),
            in_specs=[pl.BlockSpec((1, _W), index_map=lambda i: (0, i))],
            out_specs=[pl.BlockSpec((_W * D,), index_map=lambda i: (i,))],
            core_axis_name=("core", "subcore"),
            dimension_semantics=(pltpu.PARALLEL,),
        )(idx_hbm, out_hbm)

    return lookup


def kernel(x, table):
    V, D = table.shape
    B = x.size
    scale = float(np.sqrt(D).astype(np.float32))
    idx = x.reshape(1, B)
    out = _make_lookup(B, V, D, table.dtype, scale)(table.reshape(V * D), idx)
    return out.reshape(*x.shape, D)


# speedup vs baseline: 2.7805x; 2.2203x over previous
import dataclasses
import functools

import jax
import jax.numpy as jnp
import numpy as np
from jax.experimental import pallas as pl
from jax.experimental.pallas import tpu as pltpu
from jax.experimental.pallas import tpu_sc as plsc

_W = 512
_L = 16


@functools.cache
def _make_lookup(B, V, D, dtype, scale):
    mesh = plsc.VectorSubcoreMesh(core_axis_name="core", subcore_axis_name="subcore")
    cp = pltpu.CompilerParams()
    if "needs_layout_passes" in pltpu.CompilerParams.__dataclass_fields__:
        cp = dataclasses.replace(cp, needs_layout_passes=False)

    @functools.partial(
        pl.kernel,
        out_type=jax.ShapeDtypeStruct((B * D,), dtype),
        mesh=mesh,
        scratch_types=[pltpu.VMEM((V * D,), dtype)],
        compiler_params=cp,
    )
    def lookup(table_hbm, idx_hbm, out_hbm, tab_vmem):
        pltpu.sync_copy(table_hbm, tab_vmem)

        @pl.loop(0, V * D // _L)
        def _(i):
            sl = pl.ds(i * _L, _L)
            tab_vmem[sl] = tab_vmem[sl] * scale

        iotas = [jax.lax.iota(jnp.int32, _L) + h * _L for h in range(D // _L)]

        def body(i_vmem, o_vmem):
            @plsc.parallel_loop(0, _W // _L, unroll=2)
            def _(g):
                idxv = i_vmem[0, pl.ds(g * _L, _L)]
                addrs = idxv * D
                for r in range(_L):
                    sel = jnp.full((_L,), r, jnp.int32)
                    base = addrs.at[sel].get(mode="promise_in_bounds")
                    dst = g * (_L * D) + r * D
                    for h, io in enumerate(iotas):
                        vals = plsc.load_gather(tab_vmem, [base + io])
                        o_vmem[pl.ds(dst + h * _L, _L)] = vals

        pltpu.emit_pipeline(
            body,
            grid=(B // _W,),
            in_specs=[pl.BlockSpec((1, _W), index_map=lambda i: (0, i))],
            out_specs=[pl.BlockSpec((_W * D,), index_map=lambda i: (i,))],
            core_axis_name=("core", "subcore"),
            dimension_semantics=(pltpu.PARALLEL,),
        )(idx_hbm, out_hbm)

    return lookup


def kernel(x, table):
    V, D = table.shape
    B = x.size
    scale = float(np.sqrt(D).astype(np.float32))
    idx = x.reshape(1, B)
    out = _make_lookup(B, V, D, table.dtype, scale)(table.reshape(V * D), idx)
    return out.reshape(*x.shape, D)


# speedup vs baseline: 2.7978x; 1.0062x over previous
import dataclasses
import functools

import jax
import jax.numpy as jnp
import numpy as np
from jax.experimental import pallas as pl
from jax.experimental.pallas import tpu as pltpu
from jax.experimental.pallas import tpu_sc as plsc

_L = 16


@functools.cache
def _make_lookup(S, N, V, D, dtype, scale):
    mesh = plsc.VectorSubcoreMesh(core_axis_name="core", subcore_axis_name="subcore")
    cp = pltpu.CompilerParams(use_tc_tiling_on_sc=False)
    if "needs_layout_passes" in pltpu.CompilerParams.__dataclass_fields__:
        cp = dataclasses.replace(cp, needs_layout_passes=False)

    n_full = N // _L
    tail = N % _L

    @functools.partial(
        pl.kernel,
        out_type=jax.ShapeDtypeStruct((S, N, D), dtype),
        mesh=mesh,
        scratch_types=[pltpu.VMEM((V * D,), dtype)],
        compiler_params=cp,
    )
    def lookup(table_hbm, idx_hbm, out_hbm, tab_vmem):
        pltpu.sync_copy(table_hbm, tab_vmem)

        @pl.loop(0, V * D // _L)
        def _(i):
            sl = pl.ds(i * _L, _L)
            tab_vmem[sl] = tab_vmem[sl] * scale

        iotas = [jax.lax.iota(jnp.int32, _L) + h * _L for h in range(D // _L)]

        def rows16(i_vmem, o_vmem, row0):
            idxv = i_vmem[0, 0, pl.ds(row0, _L)]
            addrs = idxv * D
            for r in range(_L):
                sel = jnp.full((_L,), r, jnp.int32)
                base = addrs.at[sel].get(mode="promise_in_bounds")
                for h, io in enumerate(iotas):
                    vals = plsc.load_gather(tab_vmem, [base + io])
                    o_vmem[0, row0 + r, pl.ds(h * _L, _L)] = vals

        def body(i_vmem, o_vmem):
            @plsc.parallel_loop(0, n_full, unroll=2)
            def _(g):
                rows16(i_vmem, o_vmem, g * _L)

            if tail:
                rows16(i_vmem, o_vmem, N - _L)

        pltpu.emit_pipeline(
            body,
            grid=(S,),
            in_specs=[pl.BlockSpec((1, 1, N), index_map=lambda i: (i, 0, 0))],
            out_specs=[pl.BlockSpec((1, N, D), index_map=lambda i: (i, 0, 0))],
            core_axis_name=("core", "subcore"),
            dimension_semantics=(pltpu.PARALLEL,),
        )(idx_hbm, out_hbm)

    return lookup


def kernel(x, table):
    V, D = table.shape
    S = x.shape[0] * x.shape[1]
    N = x.shape[2]
    scale = float(np.sqrt(D).astype(np.float32))
    idx = x.reshape(S, 1, N)
    out = _make_lookup(S, N, V, D, table.dtype, scale)(table.reshape(V * D), idx)
    return out.reshape(*x.shape, D)
